# baseline (device time: 82786 ns/iter reference)
import jax
import jax.numpy as jnp
from jax import lax
from jax.experimental import pallas as pl
from jax.experimental.pallas import tpu as pltpu

N_DEV = 4
N_HOP = N_DEV - 1
N_SUB = 8
N_OSLOT = 4


def kernel(x, w_mat, scale_x, scale_w):
    m_per, k = x.shape
    n_per = w_mat.shape[1]
    half = m_per // 2
    sub = half // N_SUB

    def body(x_ref, w_ref, sx_ref, sw_ref, out_ref,
             xf_ref, wf_ref, stage_ref, wb_ref, ostage_ref, fstage_ref,
             cw_ref, ccw_ref,
             fetch_sems, w_sem, osems, fsems,
             cw_send, cw_recv, ccw_send, ccw_recv):
        my = lax.axis_index("i")
        left = lax.rem(my + N_DEV - 1, N_DEV)
        right = lax.rem(my + 1, N_DEV)

        order = [(hi, b) for b in range(N_SUB) for hi in (0, 1)]

        fetches = []
        for j, (hi, b) in enumerate(order):
            row0 = hi * half + b * sub
            cp = pltpu.make_async_copy(
                x_ref.at[pl.ds(row0, sub), :], xf_ref.at[j], fetch_sems.at[j]
            )
            cp.start()
            fetches.append(cp)
        w_cp = pltpu.make_async_copy(w_ref, wf_ref, w_sem)
        w_cp.start()

        barrier_sem = pltpu.get_barrier_semaphore()
        for nbr in (left, right):
            pl.semaphore_signal(
                barrier_sem, inc=1,
                device_id=(nbr,), device_id_type=pl.DeviceIdType.MESH,
            )
        pl.semaphore_wait(barrier_sem, 2)

        def sub_rdma(h, hi, b):
            buf = cw_ref if hi == 0 else ccw_ref
            if h == 0:
                src = stage_ref.at[hi, pl.ds(b * sub, sub), :]
            else:
                src = buf.at[h - 1, pl.ds(b * sub, sub), :]
            return pltpu.make_async_remote_copy(
                src_ref=src,
                dst_ref=buf.at[h, pl.ds(b * sub, sub), :],
                send_sem=(cw_send if hi == 0 else ccw_send).at[h, b],
                recv_sem=(cw_recv if hi == 0 else ccw_recv).at[h, b],
                device_id=(right if hi == 0 else left,),
                device_id_type=pl.DeviceIdType.MESH,
            )

        rdmas = {}
        for j, (hi, b) in enumerate(order):
            fetches[j].wait()
            stage_ref[hi, pl.ds(b * sub, sub), :] = (
                xf_ref[j].astype(jnp.float8_e4m3fn)
            )
            r = sub_rdma(0, hi, b)
            r.start()
            rdmas[(0, hi, b)] = r

        w_cp.wait()
        wb_ref[...] = wf_ref[...].astype(jnp.bfloat16)
        s = sx_ref[0] * sw_ref[0]

        n_stores = [0]
        out_cps = {}

        def silu_store(chunk_fp8, row0):
            slot = n_stores[0] % N_OSLOT
            if n_stores[0] >= N_OSLOT:
                out_cps[n_stores[0] - N_OSLOT].wait()
            acc = jnp.dot(chunk_fp8.astype(jnp.bfloat16), wb_ref[...],
                          preferred_element_type=jnp.float32)
            y = acc * s
            z = jnp.clip(y, -60.0, 60.0)
            ostage_ref[slot] = y / (1.0 + jnp.exp(-z))
            cp = pltpu.make_async_copy(
                ostage_ref.at[slot], out_ref.at[pl.ds(row0, half), :],
                osems.at[slot],
            )
            cp.start()
            out_cps[n_stores[0]] = cp
            n_stores[0] += 1

        silu_store(stage_ref[0], my * m_per)
        silu_store(stage_ref[1], my * m_per + half)

        for h in range(1, N_HOP):
            for b in range(N_SUB):
                for hi in (0, 1):
                    rdmas[(h - 1, hi, b)].wait_recv()
                    r = sub_rdma(h, hi, b)
                    r.start()
                    rdmas[(h, hi, b)] = r
            top_origin = lax.rem(my - h + N_DEV, N_DEV)
            bot_origin = lax.rem(my + h, N_DEV)
            silu_store(cw_ref[h - 1], top_origin * m_per)
            silu_store(ccw_ref[h - 1], bot_origin * m_per + half)

        f_cps = []
        for b in range(N_SUB):
            for hi in (0, 1):
                rdmas[(N_HOP - 1, hi, b)].wait_recv()
                buf = cw_ref if hi == 0 else ccw_ref
                row0 = (right if hi == 0 else left) * m_per + hi * half
                xb = buf[N_HOP - 1, pl.ds(b * sub, sub), :].astype(jnp.bfloat16)
                acc = jnp.dot(xb, wb_ref[...], preferred_element_type=jnp.float32)
                y = acc * s
                z = jnp.clip(y, -60.0, 60.0)
                fstage_ref[hi, pl.ds(b * sub, sub), :] = y / (1.0 + jnp.exp(-z))
                cp = pltpu.make_async_copy(
                    fstage_ref.at[hi, pl.ds(b * sub, sub), :],
                    out_ref.at[pl.ds(row0 + b * sub, sub), :],
                    fsems.at[hi, b],
                )
                cp.start()
                f_cps.append(cp)

        total = n_stores[0]
        for i in range(max(0, total - N_OSLOT), total):
            out_cps[i].wait()
        for cp in f_cps:
            cp.wait()
        for r in rdmas.values():
            r.wait_send()

    return pl.pallas_call(
        body,
        out_shape=jax.ShapeDtypeStruct((N_DEV * m_per, n_per), jnp.float32),
        in_specs=[
            pl.BlockSpec(memory_space=pl.ANY),
            pl.BlockSpec(memory_space=pl.ANY),
            pl.BlockSpec(memory_space=pltpu.SMEM),
            pl.BlockSpec(memory_space=pltpu.SMEM),
        ],
        out_specs=pl.BlockSpec(memory_space=pl.ANY),
        scratch_shapes=[
            pltpu.VMEM((2 * N_SUB, sub, k), jnp.float32),
            pltpu.VMEM((k, n_per), jnp.float32),
            pltpu.VMEM((2, half, k), jnp.float8_e4m3fn),
            pltpu.VMEM((k, n_per), jnp.bfloat16),
            pltpu.VMEM((N_OSLOT, half, n_per), jnp.float32),
            pltpu.VMEM((2, half, n_per), jnp.float32),
            pltpu.VMEM((N_HOP, half, k), jnp.float8_e4m3fn),
            pltpu.VMEM((N_HOP, half, k), jnp.float8_e4m3fn),
            pltpu.SemaphoreType.DMA((2 * N_SUB,)),
            pltpu.SemaphoreType.DMA,
            pltpu.SemaphoreType.DMA((N_OSLOT,)),
            pltpu.SemaphoreType.DMA((2, N_SUB)),
            pltpu.SemaphoreType.DMA((N_HOP, N_SUB)),
            pltpu.SemaphoreType.DMA((N_HOP, N_SUB)),
            pltpu.SemaphoreType.DMA((N_HOP, N_SUB)),
            pltpu.SemaphoreType.DMA((N_HOP, N_SUB)),
        ],
        compiler_params=pltpu.CompilerParams(
            collective_id=0, vmem_limit_bytes=64 * 1024 * 1024,
        ),
    )(x, w_mat, scale_x, scale_w)
